# Initial kernel scaffold; baseline (speedup 1.0000x reference)
import functools

import jax
import jax.numpy as jnp
from jax import lax
from jax.experimental import pallas as pl
from jax.experimental.pallas import tpu as pltpu
from jax.experimental.pallas import tpu_sc as plsc

_NC = 2
_NS = 16
_K = 80
_ZR = 25


def _sc_agg_body(compute_deg, G, RP, N, F, *refs):
    if compute_deg:
        (z_hbm, srccat, dstr, out_hbm, deg_hbm,
         acc, srcb, dstb, rows, zbuf, sem, degsh, onesb, zbuf16) = refs
    else:
        (z_hbm, srccat, dstr, out_hbm,
         acc, srcb, dstb, rows, zbuf, sem) = refs

    c = lax.axis_index("c")
    t = lax.axis_index("s")
    base = t * RP

    zero16 = jnp.zeros((16,), jnp.float32)

    @pl.loop(0, _ZR)
    def _(r):
        for c16 in range(F // 16):
            zbuf[r, pl.ds(c16 * 16, 16)] = zero16

    @pl.loop(0, RP // _ZR)
    def _(r):
        pltpu.sync_copy(zbuf, acc.at[pl.ds(base + r * _ZR, _ZR)])

    if compute_deg:
        one16 = jnp.full((16,), 1.0, jnp.float32)

        @pl.loop(0, _ZR)
        def _(r):
            zbuf16[r, pl.ds(0, 16)] = zero16

        @pl.loop(0, _K)
        def _(r):
            onesb[r, pl.ds(0, 16)] = one16

        @pl.loop(0, RP // _ZR)
        def _(r):
            pltpu.sync_copy(zbuf16, degsh.at[pl.ds(base + r * _ZR, _ZR)])

    plsc.subcore_barrier()

    pltpu.sync_copy(srccat.at[c, t], srcb)
    pltpu.sync_copy(dstr.at[t], dstb)

    if compute_deg:
        @pl.loop(0, G)
        def _(g):
            pltpu.async_copy(z_hbm.at[srcb.at[g]], rows, sem).wait()
            pltpu.sync_copy(rows, acc.at[dstb.at[g]], add=True)

            @pl.when(c == 0)
            def _():
                pltpu.sync_copy(onesb, degsh.at[dstb.at[g]], add=True)
    else:
        @pl.loop(0, G)
        def _(g):
            pltpu.async_copy(z_hbm.at[srcb.at[g]], rows, sem).wait()
            pltpu.sync_copy(rows, acc.at[dstb.at[g]], add=True)

    plsc.subcore_barrier()

    pltpu.sync_copy(acc.at[pl.ds(base, RP)],
                    out_hbm.at[pl.ds(c * N + base, RP)])
    if compute_deg:
        @pl.when(c == 0)
        def _():
            pltpu.sync_copy(degsh.at[pl.ds(base, RP)],
                            deg_hbm.at[pl.ds(base, RP)])


@functools.lru_cache(maxsize=None)
def _make_sc_agg(N, E, F, compute_deg):
    EP = E // _NS
    G = EP // _K
    RP = N // _NS

    mesh = plsc.VectorSubcoreMesh(core_axis_name="c", subcore_axis_name="s")
    out_type = [jax.ShapeDtypeStruct((2 * N, F), jnp.float32)]
    scratch = [
        pltpu.VMEM_SHARED((N, F), jnp.float32),
        pltpu.VMEM((G, _K), jnp.int32),
        pltpu.VMEM((G, _K), jnp.int32),
        pltpu.VMEM((_K, F), jnp.float32),
        pltpu.VMEM((_ZR, F), jnp.float32),
        pltpu.SemaphoreType.DMA,
    ]
    if compute_deg:
        out_type.append(jax.ShapeDtypeStruct((N, 16), jnp.float32))
        scratch += [
            pltpu.VMEM_SHARED((N, 16), jnp.float32),
            pltpu.VMEM((_K, 16), jnp.float32),
            pltpu.VMEM((_ZR, 16), jnp.float32),
        ]

    body = functools.partial(_sc_agg_body, compute_deg, G, RP, N, F)
    return pl.kernel(body, out_type=tuple(out_type), mesh=mesh,
                     scratch_types=tuple(scratch))


def _l1_body(x_ref, ws_ref, bs_ref, wn_ref, s_ref, z_ref):
    j = pl.program_id(1)

    @pl.when(j == 0)
    def _():
        s_ref[...] = jnp.maximum(x_ref[...] @ ws_ref[...] + bs_ref[...], 0.0)

    z_ref[...] = x_ref[...] @ wn_ref[...]


def _l2_body(s1_ref, alo_ref, ahi_ref, deg_ref, b1n_ref, ws_ref, bs_ref,
             wn_ref, s2_ref, z2_ref, n1_ref):
    j = pl.program_id(1)

    @pl.when(j == 0)
    def _():
        d = jnp.maximum(deg_ref[:, 0:1], 1.0)
        a = jnp.concatenate([alo_ref[...], ahi_ref[...]], axis=1) / d
        n1 = jnp.maximum(a + b1n_ref[...], 0.0)
        n1_ref[...] = n1
        ws = ws_ref[...]
        s2_ref[...] = jnp.maximum(
            s1_ref[...] @ ws[:256] + n1 @ ws[256:] + bs_ref[...], 0.0)

    wn = wn_ref[...]
    z2_ref[...] = s1_ref[...] @ wn[:256] + n1_ref[...] @ wn[256:]


def _l3_body(s2_ref, alo_ref, ahi_ref, deg_ref, b2n_ref, wc_ref, bc_ref,
             o_ref):
    d = jnp.maximum(deg_ref[:, 0:1], 1.0)
    a = jnp.concatenate([alo_ref[...], ahi_ref[...]], axis=1) / d
    n2 = jnp.maximum(a + b2n_ref[...], 0.0)
    s2 = s2_ref[...]
    ss = (jnp.sum(s2 * s2, axis=1, keepdims=True)
          + jnp.sum(n2 * n2, axis=1, keepdims=True))
    r = 1.0 / jnp.maximum(jnp.sqrt(ss), 1e-12)
    wc = wc_ref[...]
    o_ref[...] = (s2 * r) @ wc[:256] + (n2 * r) @ wc[256:] + bc_ref[...]


def kernel(x, edge_index, W1_self, b1_self, W1_neigh, b1_neigh,
           W2_self, b2_self, W2_neigh, b2_neigh, W_cls, b_cls):
    N, Din = x.shape
    E = edge_index.shape[1]
    H = W1_self.shape[1]
    Dout = W_cls.shape[1]
    F = H // 2
    NB = 10
    BM = N // NB
    EP = E // _NS
    G = EP // _K

    src = edge_index[0]
    dst = edge_index[1]
    srccat = jnp.concatenate([src, src + N]).reshape(_NC, _NS, G, _K)
    dstr = dst.reshape(_NS, G, _K)

    full = lambda shape: pl.BlockSpec(shape, lambda i, j: (0, 0))

    s1, z1 = pl.pallas_call(
        _l1_body,
        grid=(NB, _NC),
        in_specs=[
            pl.BlockSpec((BM, Din), lambda i, j: (i, 0)),
            full((Din, H)),
            full((1, H)),
            pl.BlockSpec((Din, F), lambda i, j: (0, j)),
        ],
        out_specs=[
            pl.BlockSpec((BM, H), lambda i, j: (i, 0)),
            pl.BlockSpec((BM, F), lambda i, j: (j * NB + i, 0)),
        ],
        out_shape=[
            jax.ShapeDtypeStruct((N, H), jnp.float32),
            jax.ShapeDtypeStruct((2 * N, F), jnp.float32),
        ],
    )(x, W1_self, b1_self.reshape(1, H), W1_neigh)

    agg1, deg16 = _make_sc_agg(N, E, F, True)(z1, srccat, dstr)

    s2, z2 = pl.pallas_call(
        _l2_body,
        grid=(NB, _NC),
        in_specs=[
            pl.BlockSpec((BM, H), lambda i, j: (i, 0)),
            pl.BlockSpec((BM, F), lambda i, j: (i, 0)),
            pl.BlockSpec((BM, F), lambda i, j: (NB + i, 0)),
            pl.BlockSpec((BM, 16), lambda i, j: (i, 0)),
            full((1, H)),
            full((2 * H, H)),
            full((1, H)),
            pl.BlockSpec((2 * H, F), lambda i, j: (0, j)),
        ],
        out_specs=[
            pl.BlockSpec((BM, H), lambda i, j: (i, 0)),
            pl.BlockSpec((BM, F), lambda i, j: (j * NB + i, 0)),
        ],
        out_shape=[
            jax.ShapeDtypeStruct((N, H), jnp.float32),
            jax.ShapeDtypeStruct((2 * N, F), jnp.float32),
        ],
        scratch_shapes=[pltpu.VMEM((BM, H), jnp.float32)],
    )(s1, agg1, agg1, deg16, b1_neigh.reshape(1, H), W2_self,
      b2_self.reshape(1, H), W2_neigh)

    (agg2,) = _make_sc_agg(N, E, F, False)(z2, srccat, dstr)

    out = pl.pallas_call(
        _l3_body,
        grid=(NB,),
        in_specs=[
            pl.BlockSpec((BM, H), lambda i: (i, 0)),
            pl.BlockSpec((BM, F), lambda i: (i, 0)),
            pl.BlockSpec((BM, F), lambda i: (NB + i, 0)),
            pl.BlockSpec((BM, 16), lambda i: (i, 0)),
            pl.BlockSpec((1, H), lambda i: (0, 0)),
            pl.BlockSpec((2 * H, Dout), lambda i: (0, 0)),
            pl.BlockSpec((1, Dout), lambda i: (0, 0)),
        ],
        out_specs=pl.BlockSpec((BM, Dout), lambda i: (i, 0)),
        out_shape=jax.ShapeDtypeStruct((N, Dout), jnp.float32),
    )(s2, agg2, agg2, deg16, b2_neigh.reshape(1, H), W_cls,
      b_cls.reshape(1, Dout))

    return out


# speedup vs baseline: 5.3669x; 5.3669x over previous
import functools

import jax
import jax.numpy as jnp
from jax import lax
from jax.experimental import pallas as pl
from jax.experimental.pallas import tpu as pltpu
from jax.experimental.pallas import tpu_sc as plsc

_NC = 2
_NS = 16
_K = 80
_KD = 40
_ZR = 24


def _zero_stripe(ref, zbuf, base, RP, N, t):
    @pl.loop(0, RP // _ZR)
    def _(r):
        pltpu.sync_copy(zbuf, ref.at[pl.ds(base + r * _ZR, _ZR)])

    tail = N - _NS * RP
    if tail:
        @pl.when(t == 0)
        def _():
            pltpu.sync_copy(zbuf.at[pl.ds(0, tail)],
                            ref.at[pl.ds(_NS * RP, tail)])


def _sc_agg_body(G, RP, N, F, z_hbm, srccat, dstr, out_hbm,
                 acc, srcb, dstb, rows, zbuf, sem):
    c = lax.axis_index("c")
    t = lax.axis_index("s")
    base = t * RP
    tail = N - _NS * RP

    zero16 = jnp.zeros((16,), jnp.float32)

    @pl.loop(0, _ZR)
    def _(r):
        for c16 in range(F // 16):
            zbuf[r, pl.ds(c16 * 16, 16)] = zero16

    _zero_stripe(acc, zbuf, base, RP, N, t)

    plsc.subcore_barrier()

    pltpu.sync_copy(srccat.at[c, t], srcb)
    pltpu.sync_copy(dstr.at[t], dstb)

    @pl.loop(0, G)
    def _(g):
        pltpu.async_copy(z_hbm.at[srcb.at[g]], rows, sem).wait()
        pltpu.sync_copy(rows, acc.at[dstb.at[g]], add=True)

    plsc.subcore_barrier()

    pltpu.sync_copy(acc.at[pl.ds(base, RP)],
                    out_hbm.at[pl.ds(c * N + base, RP)])
    if tail:
        @pl.when(t == 0)
        def _():
            pltpu.sync_copy(acc.at[pl.ds(_NS * RP, tail)],
                            out_hbm.at[pl.ds(c * N + _NS * RP, tail)])


@functools.lru_cache(maxsize=None)
def _make_sc_agg(N, E, F):
    EP = E // _NS
    G = EP // _K
    RP = (N // _NS) // _ZR * _ZR

    mesh = plsc.VectorSubcoreMesh(core_axis_name="c", subcore_axis_name="s",
                                  num_cores=_NC, num_subcores=_NS)
    scratch = (
        pltpu.VMEM_SHARED((N, F), jnp.float32),
        pltpu.VMEM((G, _K), jnp.int32),
        pltpu.VMEM((G, _K), jnp.int32),
        pltpu.VMEM((_K, F), jnp.float32),
        pltpu.VMEM((_ZR, F), jnp.float32),
        pltpu.SemaphoreType.DMA,
    )
    body = functools.partial(_sc_agg_body, G, RP, N, F)
    return pl.kernel(body,
                     out_type=jax.ShapeDtypeStruct((2 * N, F), jnp.float32),
                     mesh=mesh, scratch_types=scratch)


def _sc_deg_body(G, RP, N, F, dstr, deg_hbm, degsh, dstb, onesb, zbuf):
    c = lax.axis_index("c")
    t = lax.axis_index("s")
    base = t * RP
    tail = N - _NS * RP

    zero16 = jnp.zeros((16,), jnp.float32)
    one16 = jnp.full((16,), 1.0, jnp.float32)

    @pl.loop(0, _ZR)
    def _(r):
        for c16 in range(F // 16):
            zbuf[r, pl.ds(c16 * 16, 16)] = zero16

    @pl.loop(0, _KD)
    def _(r):
        for c16 in range(F // 16):
            onesb[r, pl.ds(c16 * 16, 16)] = one16

    _zero_stripe(degsh, zbuf, base, RP, N, t)

    plsc.subcore_barrier()

    pltpu.sync_copy(dstr.at[c, t], dstb)

    @pl.loop(0, G)
    def _(g):
        pltpu.sync_copy(onesb, degsh.at[dstb.at[g]], add=True)

    plsc.subcore_barrier()

    pltpu.sync_copy(degsh.at[pl.ds(base, RP)],
                    deg_hbm.at[pl.ds(c * N + base, RP)])
    if tail:
        @pl.when(t == 0)
        def _():
            pltpu.sync_copy(degsh.at[pl.ds(_NS * RP, tail)],
                            deg_hbm.at[pl.ds(c * N + _NS * RP, tail)])


@functools.lru_cache(maxsize=None)
def _make_sc_deg(N, E, F):
    EP = E // (_NC * _NS)
    G = EP // _KD
    RP = (N // _NS) // _ZR * _ZR

    mesh = plsc.VectorSubcoreMesh(core_axis_name="c", subcore_axis_name="s",
                                  num_cores=_NC, num_subcores=_NS)
    scratch = (
        pltpu.VMEM_SHARED((N, F), jnp.float32),
        pltpu.VMEM((G, _KD), jnp.int32),
        pltpu.VMEM((_KD, F), jnp.float32),
        pltpu.VMEM((_ZR, F), jnp.float32),
    )
    body = functools.partial(_sc_deg_body, G, RP, N, F)
    return pl.kernel(
        body,
        out_type=jax.ShapeDtypeStruct((_NC * N, F), jnp.float32),
        mesh=mesh, scratch_types=scratch)


def _l1_body(x_ref, ws_ref, bs_ref, wn_ref, s_ref, z_ref):
    j = pl.program_id(1)

    @pl.when(j == 0)
    def _():
        s_ref[...] = jnp.maximum(x_ref[...] @ ws_ref[...] + bs_ref[...], 0.0)

    z_ref[...] = x_ref[...] @ wn_ref[...]


def _l2_body(s1_ref, alo_ref, ahi_ref, dega_ref, degb_ref, b1n_ref, ws_ref,
             bs_ref, wn_ref, s2_ref, z2_ref, n1_ref):
    j = pl.program_id(1)

    @pl.when(j == 0)
    def _():
        d = jnp.maximum(dega_ref[:, 0:1] + degb_ref[:, 0:1], 1.0)
        a = jnp.concatenate([alo_ref[...], ahi_ref[...]], axis=1) / d
        n1 = jnp.maximum(a + b1n_ref[...], 0.0)
        n1_ref[...] = n1
        ws = ws_ref[...]
        s2_ref[...] = jnp.maximum(
            s1_ref[...] @ ws[:256] + n1 @ ws[256:] + bs_ref[...], 0.0)

    wn = wn_ref[...]
    z2_ref[...] = s1_ref[...] @ wn[:256] + n1_ref[...] @ wn[256:]


def _l3_body(s2_ref, alo_ref, ahi_ref, dega_ref, degb_ref, b2n_ref, wc_ref,
             bc_ref, o_ref):
    d = jnp.maximum(dega_ref[:, 0:1] + degb_ref[:, 0:1], 1.0)
    a = jnp.concatenate([alo_ref[...], ahi_ref[...]], axis=1) / d
    n2 = jnp.maximum(a + b2n_ref[...], 0.0)
    s2 = s2_ref[...]
    ss = (jnp.sum(s2 * s2, axis=1, keepdims=True)
          + jnp.sum(n2 * n2, axis=1, keepdims=True))
    r = 1.0 / jnp.maximum(jnp.sqrt(ss), 1e-12)
    wc = wc_ref[...]
    o_ref[...] = (s2 * r) @ wc[:256] + (n2 * r) @ wc[256:] + bc_ref[...]


def kernel(x, edge_index, W1_self, b1_self, W1_neigh, b1_neigh,
           W2_self, b2_self, W2_neigh, b2_neigh, W_cls, b_cls):
    N, Din = x.shape
    E = edge_index.shape[1]
    H = W1_self.shape[1]
    Dout = W_cls.shape[1]
    F = H // 2
    NB = 10
    BM = N // NB
    G = (E // _NS) // _K
    GD = (E // (_NC * _NS)) // _KD

    src = edge_index[0]
    dst = edge_index[1]
    srccat = jnp.concatenate([src, src + N]).reshape(_NC, _NS, G, _K)
    dstr = dst.reshape(_NS, G, _K)
    dstr32 = dst.reshape(_NC, _NS, GD, _KD)

    full = lambda shape: pl.BlockSpec(shape, lambda i, j: (0, 0))

    deg2 = _make_sc_deg(N, E, F)(dstr32)

    s1, z1 = pl.pallas_call(
        _l1_body,
        grid=(NB, _NC),
        in_specs=[
            pl.BlockSpec((BM, Din), lambda i, j: (i, 0)),
            full((Din, H)),
            full((1, H)),
            pl.BlockSpec((Din, F), lambda i, j: (0, j)),
        ],
        out_specs=[
            pl.BlockSpec((BM, H), lambda i, j: (i, 0)),
            pl.BlockSpec((BM, F), lambda i, j: (j * NB + i, 0)),
        ],
        out_shape=[
            jax.ShapeDtypeStruct((N, H), jnp.float32),
            jax.ShapeDtypeStruct((2 * N, F), jnp.float32),
        ],
    )(x, W1_self, b1_self.reshape(1, H), W1_neigh)

    agg1 = _make_sc_agg(N, E, F)(z1, srccat, dstr)

    s2, z2 = pl.pallas_call(
        _l2_body,
        grid=(NB, _NC),
        in_specs=[
            pl.BlockSpec((BM, H), lambda i, j: (i, 0)),
            pl.BlockSpec((BM, F), lambda i, j: (i, 0)),
            pl.BlockSpec((BM, F), lambda i, j: (NB + i, 0)),
            pl.BlockSpec((BM, F), lambda i, j: (i, 0)),
            pl.BlockSpec((BM, F), lambda i, j: (NB + i, 0)),
            full((1, H)),
            full((2 * H, H)),
            full((1, H)),
            pl.BlockSpec((2 * H, F), lambda i, j: (0, j)),
        ],
        out_specs=[
            pl.BlockSpec((BM, H), lambda i, j: (i, 0)),
            pl.BlockSpec((BM, F), lambda i, j: (j * NB + i, 0)),
        ],
        out_shape=[
            jax.ShapeDtypeStruct((N, H), jnp.float32),
            jax.ShapeDtypeStruct((2 * N, F), jnp.float32),
        ],
        scratch_shapes=[pltpu.VMEM((BM, H), jnp.float32)],
    )(s1, agg1, agg1, deg2, deg2, b1_neigh.reshape(1, H), W2_self,
      b2_self.reshape(1, H), W2_neigh)

    agg2 = _make_sc_agg(N, E, F)(z2, srccat, dstr)

    out = pl.pallas_call(
        _l3_body,
        grid=(NB,),
        in_specs=[
            pl.BlockSpec((BM, H), lambda i: (i, 0)),
            pl.BlockSpec((BM, F), lambda i: (i, 0)),
            pl.BlockSpec((BM, F), lambda i: (NB + i, 0)),
            pl.BlockSpec((BM, F), lambda i: (i, 0)),
            pl.BlockSpec((BM, F), lambda i: (NB + i, 0)),
            pl.BlockSpec((1, H), lambda i: (0, 0)),
            pl.BlockSpec((2 * H, Dout), lambda i: (0, 0)),
            pl.BlockSpec((1, Dout), lambda i: (0, 0)),
        ],
        out_specs=pl.BlockSpec((BM, Dout), lambda i: (i, 0)),
        out_shape=jax.ShapeDtypeStruct((N, Dout), jnp.float32),
    )(s2, agg2, agg2, deg2, deg2, b2_neigh.reshape(1, H), W_cls,
      b_cls.reshape(1, Dout))

    return out
